# Initial kernel scaffold; baseline (speedup 1.0000x reference)
import jax
import jax.numpy as jnp
from jax.experimental import pallas as pl


def kernel(h, edge_index):
    raise NotImplementedError("write your pallas kernel here")


# speedup vs baseline: 2.6845x; 2.6845x over previous
import functools

import jax
import jax.numpy as jnp
from jax import lax
from jax.experimental import pallas as pl
from jax.experimental.pallas import tpu as pltpu
from jax.experimental.pallas import tpu_sc as plsc

E = 320000
D = 128
N_WORKERS = 32
E_PER_W = E // N_WORKERS
CHUNK = 80
N_CHUNKS = E_PER_W // CHUNK


def _build_sc_kernel():
    mesh = plsc.VectorSubcoreMesh(core_axis_name="c", subcore_axis_name="s")

    @functools.partial(
        pl.kernel,
        out_type=jax.ShapeDtypeStruct((E,), jnp.float32),
        mesh=mesh,
        compiler_params=pltpu.CompilerParams(needs_layout_passes=False),
        scratch_types=[
            pltpu.VMEM((CHUNK,), jnp.int32),
            pltpu.VMEM((CHUNK,), jnp.int32),
            pltpu.VMEM((CHUNK, D), jnp.float32),
            pltpu.VMEM((CHUNK, D), jnp.float32),
            pltpu.VMEM((CHUNK,), jnp.float32),
            pltpu.SemaphoreType.DMA,
        ],
    )
    def sc_kernel(h_hbm, src_hbm, dst_hbm, out_hbm,
                  idx_s, idx_d, rows_s, rows_d, scores, sem):
        wid = lax.axis_index("s") * 2 + lax.axis_index("c")
        base0 = wid * E_PER_W

        def chunk_body(ci, carry):
            base = base0 + ci * CHUNK
            pltpu.sync_copy(src_hbm.at[pl.ds(base, CHUNK)], idx_s)
            pltpu.sync_copy(dst_hbm.at[pl.ds(base, CHUNK)], idx_d)
            cp_s = pltpu.async_copy(h_hbm.at[idx_s], rows_s, sem)
            cp_d = pltpu.async_copy(h_hbm.at[idx_d], rows_d, sem)
            cp_s.wait()
            cp_d.wait()

            zeros16 = jnp.zeros((16,), jnp.float32)
            for g in range(CHUNK // 16):
                scores[pl.ds(16 * g, 16)] = zeros16

            def edge_body(e, c):
                acc = rows_s[e, pl.ds(0, 16)] * rows_d[e, pl.ds(0, 16)]
                for k in range(1, D // 16):
                    acc = acc + (rows_s[e, pl.ds(16 * k, 16)]
                                 * rows_d[e, pl.ds(16 * k, 16)])
                plsc.addupdate_scatter(
                    scores, [jnp.full((16,), e, jnp.int32)], acc)
                return c

            lax.fori_loop(0, CHUNK, edge_body, 0)
            pltpu.sync_copy(scores, out_hbm.at[pl.ds(base, CHUNK)])
            return carry

        lax.fori_loop(0, N_CHUNKS, chunk_body, 0)

    return sc_kernel


_sc_kernel = _build_sc_kernel()


@jax.jit
def kernel(h, edge_index):
    src = edge_index[0].astype(jnp.int32)
    dst = edge_index[1].astype(jnp.int32)
    return _sc_kernel(h, src, dst)


# speedup vs baseline: 5.8925x; 2.1950x over previous
import functools

import jax
import jax.numpy as jnp
from jax import lax
from jax.experimental import pallas as pl
from jax.experimental.pallas import tpu as pltpu
from jax.experimental.pallas import tpu_sc as plsc

E = 320000
D = 128
N_WORKERS = 32
E_PER_W = E // N_WORKERS
CHUNK = 200
N_CHUNKS = E_PER_W // CHUNK
N_PAIRS = N_CHUNKS // 2
SC_PAD = 208


def _build_sc_kernel():
    mesh = plsc.VectorSubcoreMesh(core_axis_name="c", subcore_axis_name="s")

    @functools.partial(
        pl.kernel,
        out_type=jax.ShapeDtypeStruct((E,), jnp.float32),
        mesh=mesh,
        compiler_params=pltpu.CompilerParams(needs_layout_passes=False),
        scratch_types=[
            pltpu.VMEM((E_PER_W,), jnp.int32),
            pltpu.VMEM((E_PER_W,), jnp.int32),
            pltpu.VMEM((CHUNK, D), jnp.float32),
            pltpu.VMEM((CHUNK, D), jnp.float32),
            pltpu.VMEM((CHUNK, D), jnp.float32),
            pltpu.VMEM((CHUNK, D), jnp.float32),
            pltpu.VMEM((SC_PAD,), jnp.float32),
            pltpu.VMEM((SC_PAD,), jnp.float32),
            pltpu.SemaphoreType.DMA,
            pltpu.SemaphoreType.DMA,
            pltpu.SemaphoreType.DMA,
            pltpu.SemaphoreType.DMA,
        ],
    )
    def sc_kernel(h_hbm, src_hbm, dst_hbm, out_hbm,
                  idx_s, idx_d, rs0, rd0, rs1, rd1, sc0, sc1,
                  gsem0, gsem1, osem0, osem1):
        wid = lax.axis_index("s") * 2 + lax.axis_index("c")
        base0 = wid * E_PER_W
        pltpu.sync_copy(src_hbm.at[pl.ds(base0, E_PER_W)], idx_s)
        pltpu.sync_copy(dst_hbm.at[pl.ds(base0, E_PER_W)], idx_d)

        def fire_gather(ci, rs, rd, gsem):
            off = ci * CHUNK
            pltpu.async_copy(h_hbm.at[idx_s.at[pl.ds(off, CHUNK)]], rs, gsem)
            pltpu.async_copy(h_hbm.at[idx_d.at[pl.ds(off, CHUNK)]], rd, gsem)

        def wait_gather(ci, rs, rd, gsem):
            off = ci * CHUNK
            pltpu.make_async_copy(
                h_hbm.at[idx_s.at[pl.ds(off, CHUNK)]], rs, gsem).wait()
            pltpu.make_async_copy(
                h_hbm.at[idx_d.at[pl.ds(off, CHUNK)]], rd, gsem).wait()

        zeros16 = jnp.zeros((16,), jnp.float32)

        def compute(ci, rs, rd, scb, osem, first):
            @pl.when(jnp.logical_not(first))
            def _():
                pltpu.make_async_copy(
                    scb.at[pl.ds(0, CHUNK)],
                    out_hbm.at[pl.ds(base0 + (ci - 2) * CHUNK, CHUNK)],
                    osem).wait()
            for t in range(SC_PAD // 16):
                scb[pl.ds(16 * t, 16)] = zeros16

            @plsc.parallel_loop(0, CHUNK, 1, unroll=4)
            def _(e):
                acc = rs[e, pl.ds(0, 16)] * rd[e, pl.ds(0, 16)]
                for k in range(1, D // 16):
                    acc = acc + (rs[e, pl.ds(16 * k, 16)]
                                 * rd[e, pl.ds(16 * k, 16)])
                plsc.addupdate_scatter(
                    scb, [jnp.full((16,), e, jnp.int32)], acc)

            pltpu.async_copy(
                scb.at[pl.ds(0, CHUNK)],
                out_hbm.at[pl.ds(base0 + ci * CHUNK, CHUNK)],
                osem)

        fire_gather(0, rs0, rd0, gsem0)

        def pair_body(g, carry):
            c0 = 2 * g
            fire_gather(c0 + 1, rs1, rd1, gsem1)
            wait_gather(c0, rs0, rd0, gsem0)
            compute(c0, rs0, rd0, sc0, osem0, g == 0)

            @pl.when(g < N_PAIRS - 1)
            def _():
                fire_gather(c0 + 2, rs0, rd0, gsem0)
            wait_gather(c0 + 1, rs1, rd1, gsem1)
            compute(c0 + 1, rs1, rd1, sc1, osem1, g == 0)
            return carry

        lax.fori_loop(0, N_PAIRS, pair_body, 0)

        pltpu.make_async_copy(
            sc0.at[pl.ds(0, CHUNK)],
            out_hbm.at[pl.ds(base0 + (N_CHUNKS - 2) * CHUNK, CHUNK)],
            osem0).wait()
        pltpu.make_async_copy(
            sc1.at[pl.ds(0, CHUNK)],
            out_hbm.at[pl.ds(base0 + (N_CHUNKS - 1) * CHUNK, CHUNK)],
            osem1).wait()

    return sc_kernel


_sc_kernel = _build_sc_kernel()


@jax.jit
def kernel(h, edge_index):
    src = edge_index[0].astype(jnp.int32)
    dst = edge_index[1].astype(jnp.int32)
    return _sc_kernel(h, src, dst)


# speedup vs baseline: 5.9281x; 1.0060x over previous
import functools

import jax
import jax.numpy as jnp
from jax import lax
from jax.experimental import pallas as pl
from jax.experimental.pallas import tpu as pltpu
from jax.experimental.pallas import tpu_sc as plsc

E = 320000
D = 128
N_WORKERS = 32
E_PER_W = E // N_WORKERS
CHUNK = 200
N_CHUNKS = E_PER_W // CHUNK
N_PAIRS = N_CHUNKS // 2
SC_PAD = 208


def _build_sc_kernel():
    mesh = plsc.VectorSubcoreMesh(core_axis_name="c", subcore_axis_name="s")

    @functools.partial(
        pl.kernel,
        out_type=jax.ShapeDtypeStruct((E,), jnp.float32),
        mesh=mesh,
        compiler_params=pltpu.CompilerParams(needs_layout_passes=False),
        scratch_types=[
            pltpu.VMEM((E_PER_W,), jnp.int32),
            pltpu.VMEM((E_PER_W,), jnp.int32),
            pltpu.VMEM((CHUNK, D), jnp.float32),
            pltpu.VMEM((CHUNK, D), jnp.float32),
            pltpu.VMEM((CHUNK, D), jnp.float32),
            pltpu.VMEM((CHUNK, D), jnp.float32),
            pltpu.VMEM((SC_PAD,), jnp.float32),
            pltpu.VMEM((SC_PAD,), jnp.float32),
            pltpu.SemaphoreType.DMA,
            pltpu.SemaphoreType.DMA,
            pltpu.SemaphoreType.DMA,
            pltpu.SemaphoreType.DMA,
        ],
    )
    def sc_kernel(h_hbm, src_hbm, dst_hbm, out_hbm,
                  idx_s, idx_d, rs0, rd0, rs1, rd1, sc0, sc1,
                  gsem0, gsem1, osem0, osem1):
        wid = lax.axis_index("s") * 2 + lax.axis_index("c")
        base0 = wid * E_PER_W
        pltpu.sync_copy(src_hbm.at[pl.ds(base0, E_PER_W)], idx_s)
        pltpu.sync_copy(dst_hbm.at[pl.ds(base0, E_PER_W)], idx_d)

        def fire_gather(ci, rs, rd, gsem):
            off = ci * CHUNK
            pltpu.async_copy(h_hbm.at[idx_s.at[pl.ds(off, CHUNK)]], rs, gsem)
            pltpu.async_copy(h_hbm.at[idx_d.at[pl.ds(off, CHUNK)]], rd, gsem)

        def wait_gather(ci, rs, rd, gsem):
            off = ci * CHUNK
            pltpu.make_async_copy(
                h_hbm.at[idx_s.at[pl.ds(off, CHUNK)]], rs, gsem).wait()
            pltpu.make_async_copy(
                h_hbm.at[idx_d.at[pl.ds(off, CHUNK)]], rd, gsem).wait()

        zeros16 = jnp.zeros((16,), jnp.float32)

        def compute(ci, rs, rd, scb, osem, first):
            @pl.when(jnp.logical_not(first))
            def _():
                pltpu.make_async_copy(
                    scb.at[pl.ds(0, CHUNK)],
                    out_hbm.at[pl.ds(base0 + (ci - 2) * CHUNK, CHUNK)],
                    osem).wait()
            for t in range(SC_PAD // 16):
                scb[pl.ds(16 * t, 16)] = zeros16

            @plsc.parallel_loop(0, CHUNK, 1, unroll=8)
            def _(e):
                acc = rs[e, pl.ds(0, 16)] * rd[e, pl.ds(0, 16)]
                for k in range(1, D // 16):
                    acc = acc + (rs[e, pl.ds(16 * k, 16)]
                                 * rd[e, pl.ds(16 * k, 16)])
                plsc.addupdate_scatter(
                    scb, [jnp.full((16,), e, jnp.int32)], acc)

            pltpu.async_copy(
                scb.at[pl.ds(0, CHUNK)],
                out_hbm.at[pl.ds(base0 + ci * CHUNK, CHUNK)],
                osem)

        fire_gather(0, rs0, rd0, gsem0)

        def pair_body(g, carry):
            c0 = 2 * g
            fire_gather(c0 + 1, rs1, rd1, gsem1)
            wait_gather(c0, rs0, rd0, gsem0)
            compute(c0, rs0, rd0, sc0, osem0, g == 0)

            @pl.when(g < N_PAIRS - 1)
            def _():
                fire_gather(c0 + 2, rs0, rd0, gsem0)
            wait_gather(c0 + 1, rs1, rd1, gsem1)
            compute(c0 + 1, rs1, rd1, sc1, osem1, g == 0)
            return carry

        lax.fori_loop(0, N_PAIRS, pair_body, 0)

        pltpu.make_async_copy(
            sc0.at[pl.ds(0, CHUNK)],
            out_hbm.at[pl.ds(base0 + (N_CHUNKS - 2) * CHUNK, CHUNK)],
            osem0).wait()
        pltpu.make_async_copy(
            sc1.at[pl.ds(0, CHUNK)],
            out_hbm.at[pl.ds(base0 + (N_CHUNKS - 1) * CHUNK, CHUNK)],
            osem1).wait()

    return sc_kernel


_sc_kernel = _build_sc_kernel()


@jax.jit
def kernel(h, edge_index):
    src = edge_index[0].astype(jnp.int32)
    dst = edge_index[1].astype(jnp.int32)
    return _sc_kernel(h, src, dst)
